# initial kernel scaffold (unmeasured)
import jax
import jax.numpy as jnp
from jax import lax
from jax.experimental import pallas as pl
from jax.experimental.pallas import tpu as pltpu

N_DEV = 32


def kernel(x, w_mat):
    m_per, k = x.shape
    _, n = w_mat.shape
    n_per = n // N_DEV
    m_out = m_per * N_DEV

    def body(x_ref, w_ref, out_ref, w_buf, y_buf, copy_sems, send_sems, recv_sems):
        my_i = lax.axis_index("i")

        rdmas = []
        for d in range(N_DEV):
            slot = d % 2
            t = lax.rem(my_i + d, N_DEV)
            cp = pltpu.make_async_copy(
                w_ref.at[:, pl.ds(t * n_per, n_per)],
                w_buf.at[slot],
                copy_sems.at[slot],
            )
            cp.start()
            cp.wait()
            y = jnp.dot(x_ref[...], w_buf[slot], preferred_element_type=jnp.float32)
            y = y * jax.nn.sigmoid(y)
            if d == 0:
                out_ref[pl.ds(my_i * m_per, m_per), :] = y
            else:
                y_buf[d, :, :] = y
                rdma = pltpu.make_async_remote_copy(
                    src_ref=y_buf.at[d],
                    dst_ref=out_ref.at[pl.ds(my_i * m_per, m_per), :],
                    send_sem=send_sems.at[d],
                    recv_sem=recv_sems.at[d],
                    device_id=(t,),
                    device_id_type=pltpu.DeviceIdType.MESH,
                )
                rdma.start()
                rdmas.append(rdma)

        for rdma in rdmas:
            rdma.wait()

    return pl.pallas_call(
        body,
        out_shape=jax.ShapeDtypeStruct((m_out, n_per), jnp.float32),
        in_specs=[
            pl.BlockSpec(memory_space=pltpu.VMEM),
            pl.BlockSpec(memory_space=pltpu.ANY),
        ],
        out_specs=pl.BlockSpec(memory_space=pltpu.VMEM),
        scratch_shapes=[
            pltpu.VMEM((2, k, n_per), jnp.float32),
            pltpu.VMEM((N_DEV, m_per, n_per), jnp.float32),
            pltpu.SemaphoreType.DMA((2,)),
            pltpu.SemaphoreType.DMA((N_DEV,)),
            pltpu.SemaphoreType.DMA((N_DEV,)),
        ],
        compiler_params=pltpu.CompilerParams(collective_id=0),
    )(x, w_mat)


# baseline (device time: 93158 ns/iter reference)
import jax
import jax.numpy as jnp
from jax import lax
from jax.experimental import pallas as pl
from jax.experimental.pallas import tpu as pltpu

N_DEV = 32


def kernel(x, w_mat):
    m_per, k = x.shape
    _, n = w_mat.shape
    n_per = n // N_DEV
    m_out = m_per * N_DEV

    def body(x_ref, w_ref, out_ref, w_buf, y_buf, copy_sems, send_sems, recv_sems):
        my_i = lax.axis_index("i")

        def w_copy(d, slot):
            t = lax.rem(my_i + d, N_DEV)
            return pltpu.make_async_copy(
                w_ref.at[:, pl.ds(t * n_per, n_per)],
                w_buf.at[slot],
                copy_sems.at[slot],
            )

        w_copy(0, 0).start()

        def step(d, carry):
            slot = lax.rem(d, 2)
            w_copy(d, slot).wait()

            @pl.when(d + 1 < N_DEV)
            def _():
                w_copy(d + 1, 1 - slot).start()

            y = jnp.dot(
                x_ref[...], w_buf[slot], preferred_element_type=jnp.float32
            )
            y = y * jax.nn.sigmoid(y)

            @pl.when(d == 0)
            def _():
                out_ref[pl.ds(my_i * m_per, m_per), :] = y

            @pl.when(d != 0)
            def _():
                y_buf[pl.ds(d * m_per, m_per), :] = y
                t = lax.rem(my_i + d, N_DEV)
                rdma = pltpu.make_async_remote_copy(
                    src_ref=y_buf.at[pl.ds(d * m_per, m_per), :],
                    dst_ref=out_ref.at[pl.ds(my_i * m_per, m_per), :],
                    send_sem=send_sems.at[d],
                    recv_sem=recv_sems.at[d],
                    device_id=(t,),
                    device_id_type=pl.DeviceIdType.MESH,
                )
                rdma.start()

            return carry

        lax.fori_loop(0, N_DEV, step, 0)

        for d in range(1, N_DEV):
            t = lax.rem(my_i + d, N_DEV)
            rdma = pltpu.make_async_remote_copy(
                src_ref=y_buf.at[pl.ds(d * m_per, m_per), :],
                dst_ref=out_ref.at[pl.ds(my_i * m_per, m_per), :],
                send_sem=send_sems.at[d],
                recv_sem=recv_sems.at[d],
                device_id=(t,),
                device_id_type=pl.DeviceIdType.MESH,
            )
            rdma.wait()

    return pl.pallas_call(
        body,
        out_shape=jax.ShapeDtypeStruct((m_out, n_per), jnp.float32),
        in_specs=[
            pl.BlockSpec(memory_space=pltpu.VMEM),
            pl.BlockSpec(memory_space=pl.ANY),
        ],
        out_specs=pl.BlockSpec(memory_space=pltpu.VMEM),
        scratch_shapes=[
            pltpu.VMEM((2, k, n_per), jnp.float32),
            pltpu.VMEM((m_out, n_per), jnp.float32),
            pltpu.SemaphoreType.DMA((2,)),
            pltpu.SemaphoreType.DMA((N_DEV,)),
            pltpu.SemaphoreType.DMA((N_DEV,)),
        ],
    )(x, w_mat)


# device time: 88560 ns/iter; 1.0519x vs baseline; 1.0519x over previous
import jax
import jax.numpy as jnp
from jax import lax
from jax.experimental import pallas as pl
from jax.experimental.pallas import tpu as pltpu

N_DEV = 32
GROUP = 4
N_STEPS = N_DEV // GROUP


def kernel(x, w_mat):
    m_per, k = x.shape
    _, n = w_mat.shape
    n_per = n // N_DEV
    n_grp = n_per * GROUP
    m_out = m_per * N_DEV

    def body(x_ref, w_ref, out_ref, w_buf, stage, copy_sems, send_sems, recv_sems):
        my_i = lax.axis_index("i")
        my_grp = my_i // GROUP

        def w_copy(s, slot):
            g = lax.rem(my_grp + s, N_STEPS)
            return pltpu.make_async_copy(
                w_ref.at[:, pl.ds(g * n_grp, n_grp)],
                w_buf.at[slot],
                copy_sems.at[slot],
            )

        w_copy(0, 0).start()

        def step(s, carry):
            slot = lax.rem(s, 2)
            w_copy(s, slot).wait()

            @pl.when(s + 1 < N_STEPS)
            def _():
                w_copy(s + 1, 1 - slot).start()

            y = jnp.dot(
                x_ref[...], w_buf[slot], preferred_element_type=jnp.float32
            )
            y = y * jax.nn.sigmoid(y)
            stage[s, :, :] = y

            g = lax.rem(my_grp + s, N_STEPS)
            for j in range(GROUP):
                t = g * GROUP + j

                @pl.when(t == my_i)
                def _():
                    out_ref[pl.ds(my_i * m_per, m_per), :] = stage[
                        s, :, pl.ds(j * n_per, n_per)
                    ]

                @pl.when(t != my_i)
                def _():
                    rdma = pltpu.make_async_remote_copy(
                        src_ref=stage.at[s, :, pl.ds(j * n_per, n_per)],
                        dst_ref=out_ref.at[pl.ds(my_i * m_per, m_per), :],
                        send_sem=send_sems.at[t],
                        recv_sem=recv_sems.at[my_i],
                        device_id=(t,),
                        device_id_type=pl.DeviceIdType.MESH,
                    )
                    rdma.start()

            return carry

        lax.fori_loop(0, N_STEPS, step, 0)

        for d in range(N_DEV):

            @pl.when(d != my_i)
            def _():
                rdma = pltpu.make_async_remote_copy(
                    src_ref=stage.at[0, :, pl.ds(0, n_per)],
                    dst_ref=out_ref.at[pl.ds(d * m_per, m_per), :],
                    send_sem=send_sems.at[d],
                    recv_sem=recv_sems.at[d],
                    device_id=(d,),
                    device_id_type=pl.DeviceIdType.MESH,
                )
                rdma.wait()

    return pl.pallas_call(
        body,
        out_shape=jax.ShapeDtypeStruct((m_out, n_per), jnp.float32),
        in_specs=[
            pl.BlockSpec(memory_space=pltpu.VMEM),
            pl.BlockSpec(memory_space=pl.ANY),
        ],
        out_specs=pl.BlockSpec(memory_space=pltpu.VMEM),
        scratch_shapes=[
            pltpu.VMEM((2, k, n_grp), jnp.float32),
            pltpu.VMEM((N_STEPS, m_per, n_grp), jnp.float32),
            pltpu.SemaphoreType.DMA((2,)),
            pltpu.SemaphoreType.DMA((N_DEV,)),
            pltpu.SemaphoreType.DMA((N_DEV,)),
        ],
        compiler_params=pltpu.CompilerParams(
            vmem_limit_bytes=100 * 1024 * 1024,
        ),
    )(x, w_mat)
